# baseline (speedup 1.0000x reference)
import jax
import jax.numpy as jnp
from jax import lax
from jax.experimental import pallas as pl
from jax.experimental.pallas import tpu as pltpu
from jax.experimental.pallas import tpu_sc as plsc

_B, _T, _C = 4, 4096, 1024
_TOP_K = 8
_C_BLK = 512

_SORT8 = (
    (0, 1), (2, 3), (4, 5), (6, 7),
    (0, 2), (1, 3), (4, 6), (5, 7),
    (1, 2), (5, 6),
    (0, 4), (1, 5), (2, 6), (3, 7),
    (2, 4), (3, 5),
    (1, 2), (3, 4), (5, 6),
)

_BITONIC8 = (
    (0, 4), (1, 5), (2, 6), (3, 7),
    (0, 2), (1, 3), (4, 6), (5, 7),
    (0, 1), (2, 3), (4, 5), (6, 7),
)


def _compare_exchange(planes, net):
    planes = list(planes)
    for i, j in net:
        hi = jnp.maximum(planes[i], planes[j])
        lo = jnp.minimum(planes[i], planes[j])
        planes[i], planes[j] = hi, lo
    return planes


_CHUNK = 128
_ROWS = _CHUNK // _TOP_K


def _merge_sorted(carry, planes):
    merged = [jnp.maximum(carry[i], planes[7 - i]) for i in range(_TOP_K)]
    return _compare_exchange(merged, _BITONIC8)


def _topk_mean_kernel(conv_ref, gate_ref, out_ref):
    def load_sorted(base):
        planes = [
            (conv_ref[0, base + _ROWS * j:base + _ROWS * (j + 1), :]
             * gate_ref[0, base + _ROWS * j:base + _ROWS * (j + 1), :]
             ).astype(jnp.bfloat16)
            for j in range(_TOP_K)
        ]
        return _compare_exchange(planes, _SORT8)

    planes = load_sorted(0)
    for i in range(1, _T // _CHUNK):
        planes = _merge_sorted(planes, load_sorted(i * _CHUNK))
    planes = [p.astype(jnp.float32) for p in planes]
    r = _ROWS
    while r > 1:
        h = r // 2
        a = [p[:h, :] for p in planes]
        b = [p[h:, :] for p in planes]
        planes = [jnp.maximum(a[i], b[7 - i]) for i in range(_TOP_K)]
        planes = _compare_exchange(planes, _BITONIC8)
        r = h
    acc = planes[0]
    for p in planes[1:]:
        acc = acc + p
    out_ref[0, 0, :] = acc[0, :] * (1.0 / _TOP_K)


_SC_LANES = 16
_SC_NC, _SC_NS = 2, 16
_SC_NW = _SC_NC * _SC_NS
_SC_CG = _C // _SC_LANES
_SC_GROUPS = _B * _SC_CG
_SC_GPW = _SC_GROUPS // _SC_NW
_SC_CHUNK = 512
_SC_NCHUNK = _T // _SC_CHUNK


def _sc_rowgroup(conv_buf, gate_buf, base):
    vs = [conv_buf[base + j] * gate_buf[base + j] for j in range(_TOP_K)]
    return _compare_exchange(vs, _SORT8)


def _sc_topk_body(conv_hbm, gate_hbm, out_hbm,
                  conv0, gate0, conv1, gate1, out_buf, sem0, sem1):
    wid = lax.axis_index("s") * _SC_NC + lax.axis_index("c")
    bufs = ((conv0, gate0, sem0), (conv1, gate1, sem1))

    def issue(b, cg, ci):
        cb, gb, sem = bufs[ci % 2]
        rows = pl.ds(ci * _SC_CHUNK, _SC_CHUNK)
        return (pltpu.async_copy(conv_hbm.at[b, rows, cg, :], cb, sem),
                pltpu.async_copy(gate_hbm.at[b, rows, cg, :], gb, sem))

    for g in range(_SC_GPW):
        gg = wid * _SC_GPW + g
        b = gg // _SC_CG
        cg = gg % _SC_CG
        pend = issue(b, cg, 0)
        planes = [jnp.full((_SC_LANES,), -jnp.inf, jnp.float32)] * _TOP_K
        for ci in range(_SC_NCHUNK):
            pend[0].wait()
            pend[1].wait()
            if ci + 1 < _SC_NCHUNK:
                pend = issue(b, cg, ci + 1)
            cb, gb, _ = bufs[ci % 2]

            def body(i, carry, cb=cb, gb=gb):
                return tuple(_merge_sorted(
                    list(carry), _sc_rowgroup(cb, gb, i * _TOP_K)))

            planes = list(lax.fori_loop(
                0, _SC_CHUNK // _TOP_K, body, tuple(planes)))
        acc = planes[0]
        for p in planes[1:]:
            acc = acc + p
        out_buf[pl.ds(g * _SC_LANES, _SC_LANES)] = acc * (1.0 / _TOP_K)
    pltpu.sync_copy(
        out_buf,
        out_hbm.at[pl.ds(wid * _SC_GPW * _SC_LANES, _SC_GPW * _SC_LANES)])


def _sc_topk_pooled(local_conv, local_gate):
    local_conv = local_conv.reshape(_B, _T, _SC_CG, _SC_LANES)
    local_gate = local_gate.reshape(_B, _T, _SC_CG, _SC_LANES)
    return pl.kernel(
        _sc_topk_body,
        out_type=jax.ShapeDtypeStruct((_SC_GROUPS * _SC_LANES,), jnp.float32),
        mesh=plsc.VectorSubcoreMesh(core_axis_name="c", subcore_axis_name="s"),
        compiler_params=pltpu.CompilerParams(use_tc_tiling_on_sc=False),
        scratch_types=[
            pltpu.VMEM((_SC_CHUNK, _SC_LANES), jnp.float32),
            pltpu.VMEM((_SC_CHUNK, _SC_LANES), jnp.float32),
            pltpu.VMEM((_SC_CHUNK, _SC_LANES), jnp.float32),
            pltpu.VMEM((_SC_CHUNK, _SC_LANES), jnp.float32),
            pltpu.VMEM((_SC_GPW * _SC_LANES,), jnp.float32),
            pltpu.SemaphoreType.DMA,
            pltpu.SemaphoreType.DMA,
        ],
    )(local_conv, local_gate)


def _dense_kernel(pooled_ref, w_ref, b_ref, out_ref):
    acc = jnp.dot(pooled_ref[...], w_ref[...],
                  preferred_element_type=jnp.float32)
    out_ref[...] = jnp.maximum(acc + b_ref[...], 0.0)


def kernel(local_conv, local_gate, W, b):
    pooled = _sc_topk_pooled(local_conv, local_gate).reshape(_B, _C)

    out = pl.pallas_call(
        _dense_kernel,
        in_specs=[
            pl.BlockSpec((_B, _C), lambda: (0, 0)),
            pl.BlockSpec((_C, _C), lambda: (0, 0)),
            pl.BlockSpec((_C,), lambda: (0,)),
        ],
        out_specs=pl.BlockSpec((_B, _C), lambda: (0, 0)),
        out_shape=jax.ShapeDtypeStruct((_B, _C), jnp.float32),
    )(pooled, W, b)
    return out


def _tc_kernel(local_conv, local_gate, W, b):
    pooled = pl.pallas_call(
        _topk_mean_kernel,
        grid=(_B, _C // _C_BLK),
        in_specs=[
            pl.BlockSpec((1, _T, _C_BLK), lambda i, j: (i, 0, j)),
            pl.BlockSpec((1, _T, _C_BLK), lambda i, j: (i, 0, j)),
        ],
        out_specs=pl.BlockSpec((1, 1, _C_BLK), lambda i, j: (i, 0, j)),
        out_shape=jax.ShapeDtypeStruct((_B, 1, _C), jnp.float32),
    )(local_conv, local_gate)
    pooled = pooled.reshape(_B, _C)

    out = pl.pallas_call(
        _dense_kernel,
        in_specs=[
            pl.BlockSpec((_B, _C), lambda: (0, 0)),
            pl.BlockSpec((_C, _C), lambda: (0, 0)),
            pl.BlockSpec((_C,), lambda: (0,)),
        ],
        out_specs=pl.BlockSpec((_B, _C), lambda: (0, 0)),
        out_shape=jax.ShapeDtypeStruct((_B, _C), jnp.float32),
    )(pooled, W, b)
    return out


# speedup vs baseline: 22.4587x; 22.4587x over previous
import jax
import jax.numpy as jnp
from jax import lax
from jax.experimental import pallas as pl
from jax.experimental.pallas import tpu as pltpu
from jax.experimental.pallas import tpu_sc as plsc

_B, _T, _C = 4, 4096, 1024
_TOP_K = 8
_C_BLK = 256

_SORT8 = (
    (0, 1), (2, 3), (4, 5), (6, 7),
    (0, 2), (1, 3), (4, 6), (5, 7),
    (1, 2), (5, 6),
    (0, 4), (1, 5), (2, 6), (3, 7),
    (2, 4), (3, 5),
    (1, 2), (3, 4), (5, 6),
)

_BITONIC8 = (
    (0, 4), (1, 5), (2, 6), (3, 7),
    (0, 2), (1, 3), (4, 6), (5, 7),
    (0, 1), (2, 3), (4, 5), (6, 7),
)


def _compare_exchange(planes, net):
    planes = list(planes)
    for i, j in net:
        hi = jnp.maximum(planes[i], planes[j])
        lo = jnp.minimum(planes[i], planes[j])
        planes[i], planes[j] = hi, lo
    return planes


_CHUNK = 128
_ROWS = _CHUNK // _TOP_K


def _merge_sorted(carry, planes):
    merged = [jnp.maximum(carry[i], planes[7 - i]) for i in range(_TOP_K)]
    return _compare_exchange(merged, _BITONIC8)


_T_BLK = 1024


def _topk_mean_kernel(conv_ref, gate_ref, out_ref):
    for c0 in range(0, _C, _C_BLK):
        def load_sorted(base, c0=c0):
            planes = [
                (conv_ref[0, base + _ROWS * j:base + _ROWS * (j + 1),
                          c0:c0 + _C_BLK]
                 * gate_ref[0, base + _ROWS * j:base + _ROWS * (j + 1),
                            c0:c0 + _C_BLK]
                 ).astype(jnp.bfloat16)
                for j in range(_TOP_K)
            ]
            return _compare_exchange(planes, _SORT8)

        planes = load_sorted(0)
        for i in range(1, _T_BLK // _CHUNK):
            planes = _merge_sorted(planes, load_sorted(i * _CHUNK))
        planes = [p.astype(jnp.float32) for p in planes]
        r = _ROWS
        while r > 1:
            h = r // 2
            a = [p[:h, :] for p in planes]
            b = [p[h:, :] for p in planes]
            planes = [jnp.maximum(a[i], b[7 - i]) for i in range(_TOP_K)]
            planes = _compare_exchange(planes, _BITONIC8)
            r = h
        for j in range(_TOP_K):
            out_ref[0, 0, j, c0:c0 + _C_BLK] = planes[j][0, :]


_SC_LANES = 16
_SC_NC, _SC_NS = 2, 16
_SC_NW = _SC_NC * _SC_NS
_SC_CG = _C // _SC_LANES
_SC_GROUPS = _B * _SC_CG
_SC_GPW = _SC_GROUPS // _SC_NW
_SC_CHUNK = 512
_SC_NCHUNK = _T // _SC_CHUNK


def _sc_rowgroup(conv_buf, gate_buf, base):
    vs = [conv_buf[base + j] * gate_buf[base + j] for j in range(_TOP_K)]
    return _compare_exchange(vs, _SORT8)


def _sc_topk_body(conv_hbm, gate_hbm, out_hbm,
                  conv0, gate0, conv1, gate1, out_buf, sem0, sem1):
    wid = lax.axis_index("s") * _SC_NC + lax.axis_index("c")
    bufs = ((conv0, gate0, sem0), (conv1, gate1, sem1))

    def issue(b, cg, ci):
        cb, gb, sem = bufs[ci % 2]
        rows = pl.ds(ci * _SC_CHUNK, _SC_CHUNK)
        return (pltpu.async_copy(conv_hbm.at[b, rows, cg, :], cb, sem),
                pltpu.async_copy(gate_hbm.at[b, rows, cg, :], gb, sem))

    for g in range(_SC_GPW):
        gg = wid * _SC_GPW + g
        b = gg // _SC_CG
        cg = gg % _SC_CG
        pend = issue(b, cg, 0)
        planes = [jnp.full((_SC_LANES,), -jnp.inf, jnp.float32)] * _TOP_K
        for ci in range(_SC_NCHUNK):
            pend[0].wait()
            pend[1].wait()
            if ci + 1 < _SC_NCHUNK:
                pend = issue(b, cg, ci + 1)
            cb, gb, _ = bufs[ci % 2]

            def body(i, carry, cb=cb, gb=gb):
                return tuple(_merge_sorted(
                    list(carry), _sc_rowgroup(cb, gb, i * _TOP_K)))

            planes = list(lax.fori_loop(
                0, _SC_CHUNK // _TOP_K, body, tuple(planes)))
        acc = planes[0]
        for p in planes[1:]:
            acc = acc + p
        out_buf[pl.ds(g * _SC_LANES, _SC_LANES)] = acc * (1.0 / _TOP_K)
    pltpu.sync_copy(
        out_buf,
        out_hbm.at[pl.ds(wid * _SC_GPW * _SC_LANES, _SC_GPW * _SC_LANES)])


def _sc_topk_pooled(local_conv, local_gate):
    local_conv = local_conv.reshape(_B, _T, _SC_CG, _SC_LANES)
    local_gate = local_gate.reshape(_B, _T, _SC_CG, _SC_LANES)
    return pl.kernel(
        _sc_topk_body,
        out_type=jax.ShapeDtypeStruct((_SC_GROUPS * _SC_LANES,), jnp.float32),
        mesh=plsc.VectorSubcoreMesh(core_axis_name="c", subcore_axis_name="s"),
        compiler_params=pltpu.CompilerParams(use_tc_tiling_on_sc=False),
        scratch_types=[
            pltpu.VMEM((_SC_CHUNK, _SC_LANES), jnp.float32),
            pltpu.VMEM((_SC_CHUNK, _SC_LANES), jnp.float32),
            pltpu.VMEM((_SC_CHUNK, _SC_LANES), jnp.float32),
            pltpu.VMEM((_SC_CHUNK, _SC_LANES), jnp.float32),
            pltpu.VMEM((_SC_GPW * _SC_LANES,), jnp.float32),
            pltpu.SemaphoreType.DMA,
            pltpu.SemaphoreType.DMA,
        ],
    )(local_conv, local_gate)


def _dense_kernel(shards_ref, w_ref, b_ref, out_ref):
    nshard = _T // _T_BLK
    shard_planes = [
        [shards_ref[:, s, j, :] for j in range(_TOP_K)]
        for s in range(nshard)
    ]
    while len(shard_planes) > 1:
        shard_planes = [
            _merge_sorted(shard_planes[2 * i], shard_planes[2 * i + 1])
            for i in range(len(shard_planes) // 2)
        ]
    planes = shard_planes[0]
    acc = planes[0]
    for p in planes[1:]:
        acc = acc + p
    pooled = acc * (1.0 / _TOP_K)
    out = jnp.dot(pooled, w_ref[...], preferred_element_type=jnp.float32)
    out_ref[...] = jnp.maximum(out + b_ref[...], 0.0)


def kernel(local_conv, local_gate, W, b):
    nshard = _T // _T_BLK
    shards = pl.pallas_call(
        _topk_mean_kernel,
        grid=(_B, nshard),
        in_specs=[
            pl.BlockSpec((1, _T_BLK, _C), lambda i, k: (i, k, 0)),
            pl.BlockSpec((1, _T_BLK, _C), lambda i, k: (i, k, 0)),
        ],
        out_specs=pl.BlockSpec((1, 1, _TOP_K, _C), lambda i, k: (i, k, 0, 0)),
        out_shape=jax.ShapeDtypeStruct((_B, nshard, _TOP_K, _C), jnp.float32),
    )(local_conv, local_gate)

    out = pl.pallas_call(
        _dense_kernel,
        in_specs=[
            pl.BlockSpec((_B, nshard, _TOP_K, _C), lambda: (0, 0, 0, 0)),
            pl.BlockSpec((_C, _C), lambda: (0, 0)),
            pl.BlockSpec((_C,), lambda: (0,)),
        ],
        out_specs=pl.BlockSpec((_B, _C), lambda: (0, 0)),
        out_shape=jax.ShapeDtypeStruct((_B, _C), jnp.float32),
    )(shards, W, b)
    return out
